# Initial kernel scaffold; baseline (speedup 1.0000x reference)
import functools

import jax
import jax.numpy as jnp
from jax import lax
from jax.experimental import pallas as pl
from jax.experimental.pallas import tpu as pltpu
from jax.experimental.pallas import tpu_sc as plsc

NUM_EMB = 1000000
DIM = 64
ROWS = 16384
COLS = 50
B_TOTAL = ROWS * COLS
NUM_CORES = 2
NUM_SUBCORES = 16
NW = NUM_CORES * NUM_SUBCORES
CHUNK = 128
N_CHUNKS = B_TOTAL // (NW * CHUNK)
NBUF = 4

_mesh = plsc.VectorSubcoreMesh(
    core_axis_name="c", subcore_axis_name="s",
    num_cores=NUM_CORES, num_subcores=NUM_SUBCORES)


@functools.partial(
    pl.kernel,
    mesh=_mesh,
    out_type=jax.ShapeDtypeStruct((B_TOTAL, DIM), jnp.float32),
    scratch_types=[
        pltpu.VMEM((N_CHUNKS, CHUNK), jnp.int32),
        pltpu.VMEM((NBUF, CHUNK, DIM), jnp.float32),
    ] + [pltpu.SemaphoreType.DMA] * (2 * NBUF),
)
def _embed_sc(idx_hbm, table_hbm, out_hbm, idx_v, rows_v, *sems):
    gsem = sems[:NBUF]
    osem = sems[NBUF:]
    wid = lax.axis_index("s") * NUM_CORES + lax.axis_index("c")
    chunk0 = wid * N_CHUNKS

    pltpu.sync_copy(idx_hbm.at[pl.ds(chunk0, N_CHUNKS)], idx_v)

    def fire_gather(g, b):
        pltpu.async_copy(table_hbm.at[idx_v.at[g]], rows_v.at[b], gsem[b])

    def wait_gather(g, b):
        pltpu.make_async_copy(table_hbm.at[idx_v.at[g]], rows_v.at[b],
                              gsem[b]).wait()

    def out_slice(g):
        return out_hbm.at[pl.ds((chunk0 + g) * CHUNK, CHUNK)]

    for b in range(NBUF):
        fire_gather(b, b)

    def body(q, carry):
        for b in range(NBUF):
            g = q * NBUF + b
            wait_gather(g, b)
            pltpu.async_copy(rows_v.at[b], out_slice(g), osem[b])
            pltpu.make_async_copy(rows_v.at[b], out_slice(g), osem[b]).wait()
            fire_gather(g + NBUF, b)
        return carry

    lax.fori_loop(0, N_CHUNKS // NBUF - 1, body, 0)

    for b in range(NBUF):
        g = N_CHUNKS - NBUF + b
        wait_gather(g, b)
        pltpu.async_copy(rows_v.at[b], out_slice(g), osem[b])
        pltpu.make_async_copy(rows_v.at[b], out_slice(g), osem[b]).wait()


def kernel(input_, weight):
    idx = input_.reshape(-1).astype(jnp.int32).reshape(NW * N_CHUNKS, CHUNK)
    out = _embed_sc(idx, weight)
    return out.reshape(ROWS, COLS, DIM)


# speedup vs baseline: 1.8775x; 1.8775x over previous
import functools

import jax
import jax.numpy as jnp
from jax import lax
from jax.experimental import pallas as pl
from jax.experimental.pallas import tpu as pltpu
from jax.experimental.pallas import tpu_sc as plsc

NUM_EMB = 1000000
DIM = 64
ROWS = 16384
COLS = 50
B_TOTAL = ROWS * COLS
NUM_CORES = 2
NUM_SUBCORES = 16
NW = NUM_CORES * NUM_SUBCORES
CHUNK = 128
N_CHUNKS = B_TOTAL // (NW * CHUNK)
NBUF = 4

_mesh = plsc.VectorSubcoreMesh(
    core_axis_name="c", subcore_axis_name="s",
    num_cores=NUM_CORES, num_subcores=NUM_SUBCORES)


@functools.partial(
    pl.kernel,
    mesh=_mesh,
    out_type=jax.ShapeDtypeStruct((B_TOTAL, DIM), jnp.float32),
    scratch_types=[
        pltpu.VMEM((N_CHUNKS, CHUNK), jnp.int32),
        pltpu.VMEM((NBUF, CHUNK, DIM), jnp.float32),
    ] + [pltpu.SemaphoreType.DMA] * (2 * NBUF),
    compiler_params=pltpu.CompilerParams(use_tc_tiling_on_sc=False),
)
def _embed_sc(idx_hbm, table_hbm, out_hbm, idx_v, rows_v, *sems):
    gsem = sems[:NBUF]
    osem = sems[NBUF:]
    wid = lax.axis_index("s") * NUM_CORES + lax.axis_index("c")
    chunk0 = wid * N_CHUNKS

    pltpu.sync_copy(idx_hbm.at[pl.ds(chunk0, N_CHUNKS)], idx_v)

    def fire_gather(g, b):
        pltpu.async_copy(table_hbm.at[idx_v.at[g]], rows_v.at[b], gsem[b])

    def wait_gather(g, b):
        pltpu.make_async_copy(table_hbm.at[idx_v.at[g]], rows_v.at[b],
                              gsem[b]).wait()

    def out_slice(g):
        return out_hbm.at[pl.ds((chunk0 + g) * CHUNK, CHUNK)]

    for b in range(NBUF):
        fire_gather(b, b)

    def body(q, carry):
        for b in range(NBUF):
            g = q * NBUF + b
            wait_gather(g, b)
            pltpu.async_copy(rows_v.at[b], out_slice(g), osem[b])
            pltpu.make_async_copy(rows_v.at[b], out_slice(g), osem[b]).wait()
            fire_gather(g + NBUF, b)
        return carry

    lax.fori_loop(0, N_CHUNKS // NBUF - 1, body, 0)

    for b in range(NBUF):
        g = N_CHUNKS - NBUF + b
        wait_gather(g, b)
        pltpu.async_copy(rows_v.at[b], out_slice(g), osem[b])
        pltpu.make_async_copy(rows_v.at[b], out_slice(g), osem[b]).wait()


def kernel(input_, weight):
    idx = input_.reshape(-1).astype(jnp.int32).reshape(NW * N_CHUNKS, CHUNK)
    out = _embed_sc(idx, weight)
    return out.reshape(ROWS, COLS, DIM)
